# Initial kernel scaffold; baseline (speedup 1.0000x reference)
import functools

import numpy as np
import jax
import jax.numpy as jnp
from jax import lax
from jax.experimental import pallas as pl
from jax.experimental.pallas import tpu as pltpu
from jax.experimental.pallas import tpu_sc as plsc

N = 10000
NP = 10240
E = 320000
IN_C = 128
HID = 256
HALF = 128
OUT_C = 128
L = 4
ALPHA = 0.1
THETA = 0.5
BN_SCALE = float(1.0 / np.sqrt(1.0 + 1e-5))

NC, NS = 2, 16
NW = NC * NS
RPT = NP // NS
CH = 128

_MESH = dict(core_axis_name="c", subcore_axis_name="s", num_cores=NC,
             num_subcores=NS)


_EPW = E // NW
_DCH_FULL = _EPW // CH
_DTAIL = _EPW - _DCH_FULL * CH


def _deg_body(col_hbm, out_hbm, cidx, ones_v, zeros_v, deg_sh, cisem):
    c = lax.axis_index("c")
    s = lax.axis_index("s")
    wid = s * NC + c
    ebase = wid * _EPW
    rb = s * RPT

    def fill(ref, n, val):
        def body(i, _):
            ref[pl.ds(i * 16, 16)] = jnp.full((16,), val, jnp.float32)
            return _
        lax.fori_loop(0, n // 16, body, 0)

    fill(ones_v, CH, 1.0)
    fill(zeros_v, RPT, 0.0)
    pltpu.sync_copy(zeros_v, deg_sh.at[pl.ds(rb, RPT)])
    plsc.subcore_barrier()

    pltpu.async_copy(col_hbm.at[pl.ds(ebase, CH)], cidx.at[0], cisem.at[0])

    def pair(m, _):
        for sl in range(2):
            j = 2 * m + sl
            off = ebase + j * CH
            pltpu.make_async_copy(
                col_hbm.at[pl.ds(off, CH)], cidx.at[sl], cisem.at[sl]).wait()
            nxt = jnp.minimum(j + 1, _DCH_FULL - 1)
            pltpu.async_copy(col_hbm.at[pl.ds(ebase + nxt * CH, CH)],
                             cidx.at[1 - sl], cisem.at[1 - sl])
            pltpu.sync_copy(ones_v, deg_sh.at[cidx.at[sl]], add=True)
        return _

    lax.fori_loop(0, _DCH_FULL // 2, pair, 0)
    pltpu.make_async_copy(col_hbm.at[pl.ds(ebase + (_DCH_FULL - 1) * CH, CH)],
                          cidx.at[0], cisem.at[0]).wait()
    pltpu.sync_copy(col_hbm.at[pl.ds(ebase + _DCH_FULL * CH, _DTAIL)],
                    cidx.at[0, pl.ds(0, _DTAIL)])
    pltpu.sync_copy(ones_v.at[pl.ds(0, _DTAIL)],
                    deg_sh.at[cidx.at[0, pl.ds(0, _DTAIL)]], add=True)

    plsc.subcore_barrier()
    pltpu.sync_copy(deg_sh.at[pl.ds(rb, RPT)], out_hbm.at[c, pl.ds(rb, RPT)])


_deg_call = functools.partial(
    pl.kernel,
    out_type=jax.ShapeDtypeStruct((NC, NP), jnp.float32),
    mesh=plsc.VectorSubcoreMesh(**_MESH),
    scratch_types=[
        pltpu.VMEM((2, CH), jnp.int32),
        pltpu.VMEM((CH,), jnp.float32),
        pltpu.VMEM((RPT,), jnp.float32),
        pltpu.VMEM_SHARED((NP,), jnp.float32),
        pltpu.SemaphoreType.DMA((2,)),
    ],
)(_deg_body)


_EPT = E // NS
G = 6
_NCH = _EPT // CH
_NG = _NCH // G
_PTAIL = _EPT - _NCH * CH


def _prop_body(hs_hbm, row_hbm, col_hbm, out_hbm,
               ridx, cidx, rows, agg_sh, gsem, ssem, risem, cisem):
    c = lax.axis_index("c")
    s = lax.axis_index("s")
    ebase = s * _EPT
    rb = s * RPT
    hs_c = hs_hbm.at[c]

    pltpu.sync_copy(hs_hbm.at[c, pl.ds(rb, RPT)], agg_sh.at[pl.ds(rb, RPT)])
    plsc.subcore_barrier()

    def group(g, _):
        base = ebase + g * (G * CH)

        @pl.when(g > 0)
        def _wait_prev():
            for i in range(G):
                pltpu.make_async_copy(
                    rows.at[i], agg_sh.at[cidx.at[i]], ssem.at[i]).wait()

        for i in range(G):
            off = base + i * CH
            pltpu.async_copy(row_hbm.at[pl.ds(off, CH)], ridx.at[i],
                             risem.at[i])
            pltpu.async_copy(col_hbm.at[pl.ds(off, CH)], cidx.at[i],
                             cisem.at[i])
        for i in range(G):
            off = base + i * CH
            pltpu.make_async_copy(row_hbm.at[pl.ds(off, CH)], ridx.at[i],
                                  risem.at[i]).wait()
            pltpu.async_copy(hs_c.at[ridx.at[i]], rows.at[i], gsem.at[i])
        for i in range(G):
            off = base + i * CH
            pltpu.make_async_copy(col_hbm.at[pl.ds(off, CH)], cidx.at[i],
                                  cisem.at[i]).wait()
            pltpu.make_async_copy(hs_c.at[ridx.at[i]], rows.at[i],
                                  gsem.at[i]).wait()
            pltpu.async_copy(rows.at[i], agg_sh.at[cidx.at[i]], ssem.at[i],
                             add=True)
        return _

    lax.fori_loop(0, _NG, group, 0)
    for i in range(G):
        pltpu.make_async_copy(rows.at[i], agg_sh.at[cidx.at[i]],
                              ssem.at[i]).wait()

    toff = ebase + _NCH * CH
    pltpu.sync_copy(row_hbm.at[pl.ds(toff, _PTAIL)],
                    ridx.at[0, pl.ds(0, _PTAIL)])
    pltpu.sync_copy(col_hbm.at[pl.ds(toff, _PTAIL)],
                    cidx.at[0, pl.ds(0, _PTAIL)])
    pltpu.async_copy(hs_c.at[ridx.at[0, pl.ds(0, _PTAIL)]],
                     rows.at[0, pl.ds(0, _PTAIL)], gsem.at[0]).wait()
    pltpu.sync_copy(rows.at[0, pl.ds(0, _PTAIL)],
                    agg_sh.at[cidx.at[0, pl.ds(0, _PTAIL)]], add=True)

    plsc.subcore_barrier()
    pltpu.sync_copy(agg_sh.at[pl.ds(rb, RPT)], out_hbm.at[c, pl.ds(rb, RPT)])


_prop_call = functools.partial(
    pl.kernel,
    out_type=jax.ShapeDtypeStruct((NC, NP, HALF), jnp.float32),
    mesh=plsc.VectorSubcoreMesh(**_MESH),
    scratch_types=[
        pltpu.VMEM((G, CH), jnp.int32),
        pltpu.VMEM((G, CH), jnp.int32),
        pltpu.VMEM((G, CH, HALF), jnp.float32),
        pltpu.VMEM_SHARED((NP, HALF), jnp.float32),
        pltpu.SemaphoreType.DMA((G,)),
        pltpu.SemaphoreType.DMA((G,)),
        pltpu.SemaphoreType.DMA((G,)),
        pltpu.SemaphoreType.DMA((G,)),
    ],
)(_prop_body)


_R = 2048


def _stage1_body(x_ref, w0_ref, b0_ref, d0_ref, d1_ref,
                 x0_ref, hs_ref, dinv_ref):
    d = d0_ref[...] + d1_ref[...] + 1.0
    dinv = lax.rsqrt(d)
    h = jnp.maximum(
        jnp.dot(x_ref[...], w0_ref[...], preferred_element_type=jnp.float32)
        + b0_ref[...], 0.0)
    x0_ref[...] = h
    dinv_ref[...] = dinv
    hs = h * dinv
    hs_ref[0] = hs[:, :HALF]
    hs_ref[1] = hs[:, HALF:]


def _stage1(x_pad, W0, b0, d0, d1):
    return pl.pallas_call(
        _stage1_body,
        grid=(NP // _R,),
        in_specs=[
            pl.BlockSpec((_R, IN_C), lambda i: (i, 0)),
            pl.BlockSpec((IN_C, HID), lambda i: (0, 0)),
            pl.BlockSpec((1, HID), lambda i: (0, 0)),
            pl.BlockSpec((_R, 1), lambda i: (i, 0)),
            pl.BlockSpec((_R, 1), lambda i: (i, 0)),
        ],
        out_specs=[
            pl.BlockSpec((_R, HID), lambda i: (i, 0)),
            pl.BlockSpec((NC, _R, HALF), lambda i: (0, i, 0)),
            pl.BlockSpec((_R, 1), lambda i: (i, 0)),
        ],
        out_shape=[
            jax.ShapeDtypeStruct((NP, HID), jnp.float32),
            jax.ShapeDtypeStruct((NC, NP, HALF), jnp.float32),
            jax.ShapeDtypeStruct((NP, 1), jnp.float32),
        ],
    )(x_pad, W0, b0, d0, d1)


def _layer_body(beta, final, a_ref, x0_ref, dinv_ref, w_ref, g_ref, bb_ref,
                w1_ref, b1_ref, o_ref):
    dinv = dinv_ref[...]
    aggs = jnp.concatenate([a_ref[0], a_ref[1]], axis=1)
    t = (1.0 - ALPHA) * (aggs * dinv) + ALPHA * x0_ref[...]
    u = (1.0 - beta) * t + beta * jnp.dot(
        t, w_ref[...], preferred_element_type=jnp.float32)
    h = jnp.maximum(u * (BN_SCALE * g_ref[...]) + bb_ref[...], 0.0)
    if final:
        o_ref[...] = jnp.dot(
            h, w1_ref[...], preferred_element_type=jnp.float32) + b1_ref[...]
    else:
        hs = h * dinv
        o_ref[0] = hs[:, :HALF]
        o_ref[1] = hs[:, HALF:]


def _layer(i, aggs, x0, dinv, W, gamma, bnb, W1, b1):
    beta = float(np.log(THETA / (i + 1) + 1.0))
    final = i == L - 1
    if final:
        out_spec = pl.BlockSpec((_R, OUT_C), lambda i: (i, 0))
        out_shape = jax.ShapeDtypeStruct((NP, OUT_C), jnp.float32)
    else:
        out_spec = pl.BlockSpec((NC, _R, HALF), lambda i: (0, i, 0))
        out_shape = jax.ShapeDtypeStruct((NC, NP, HALF), jnp.float32)
    return pl.pallas_call(
        functools.partial(_layer_body, beta, final),
        grid=(NP // _R,),
        in_specs=[
            pl.BlockSpec((NC, _R, HALF), lambda i: (0, i, 0)),
            pl.BlockSpec((_R, HID), lambda i: (i, 0)),
            pl.BlockSpec((_R, 1), lambda i: (i, 0)),
            pl.BlockSpec((HID, HID), lambda i: (0, 0)),
            pl.BlockSpec((1, HID), lambda i: (0, 0)),
            pl.BlockSpec((1, HID), lambda i: (0, 0)),
            pl.BlockSpec((HID, OUT_C), lambda i: (0, 0)),
            pl.BlockSpec((1, OUT_C), lambda i: (0, 0)),
        ],
        out_specs=out_spec,
        out_shape=out_shape,
    )(aggs, x0, dinv, W, gamma, bnb, W1, b1)


@jax.jit
def kernel(x, edge_index, W0, b0, W1, b1, conv_ws, bn_gamma, bn_beta):
    row = edge_index[0]
    col = edge_index[1]
    x_pad = jnp.pad(x, ((0, NP - N), (0, 0)))

    deg = _deg_call(col)
    d0 = deg[0].reshape(NP, 1)
    d1 = deg[1].reshape(NP, 1)

    x0, hs, dinv = _stage1(x_pad, W0, b0.reshape(1, HID), d0, d1)

    b1r = b1.reshape(1, OUT_C)
    res = None
    for i in range(L):
        aggs = _prop_call(hs, row, col)
        res = _layer(i, aggs, x0, dinv, conv_ws[i],
                     bn_gamma[i].reshape(1, HID), bn_beta[i].reshape(1, HID),
                     W1, b1r)
        hs = res
    return res[:N]


# speedup vs baseline: 13.5615x; 13.5615x over previous
import functools

import numpy as np
import jax
import jax.numpy as jnp
from jax import lax
from jax.experimental import pallas as pl
from jax.experimental.pallas import tpu as pltpu
from jax.experimental.pallas import tpu_sc as plsc

N = 10000
NP = 10240
E = 320000
IN_C = 128
HID = 256
HALF = 128
OUT_C = 128
L = 4
ALPHA = 0.1
THETA = 0.5
BN_SCALE = float(1.0 / np.sqrt(1.0 + 1e-5))

NC, NS = 2, 16
NW = NC * NS
RPT = NP // NS
CH = 128

_MESH = dict(core_axis_name="c", subcore_axis_name="s", num_cores=NC,
             num_subcores=NS)


_EPW = E // NW
_DCH_FULL = _EPW // CH
_DTAIL = _EPW - _DCH_FULL * CH


def _deg_body(col_hbm, out_hbm, cidx, ones_v, zeros_v, deg_sh, cisem):
    c = lax.axis_index("c")
    s = lax.axis_index("s")
    wid = s * NC + c
    ebase = wid * _EPW
    rb = s * RPT

    def fill(ref, n, val):
        def body(i, _):
            ref[pl.ds(i * 16, 16)] = jnp.full((16,), val, jnp.float32)
            return _
        lax.fori_loop(0, n // 16, body, 0)

    fill(ones_v, CH, 1.0)
    fill(zeros_v, RPT, 0.0)
    pltpu.sync_copy(zeros_v, deg_sh.at[pl.ds(rb, RPT)])
    plsc.subcore_barrier()

    pltpu.async_copy(col_hbm.at[pl.ds(ebase, CH)], cidx.at[0], cisem.at[0])

    def pair(m, _):
        for sl in range(2):
            j = 2 * m + sl
            off = ebase + j * CH
            pltpu.make_async_copy(
                col_hbm.at[pl.ds(off, CH)], cidx.at[sl], cisem.at[sl]).wait()
            nxt = jnp.minimum(j + 1, _DCH_FULL - 1)
            pltpu.async_copy(col_hbm.at[pl.ds(ebase + nxt * CH, CH)],
                             cidx.at[1 - sl], cisem.at[1 - sl])
            pltpu.sync_copy(ones_v, deg_sh.at[cidx.at[sl]], add=True)
        return _

    lax.fori_loop(0, _DCH_FULL // 2, pair, 0)
    pltpu.make_async_copy(col_hbm.at[pl.ds(ebase + (_DCH_FULL - 1) * CH, CH)],
                          cidx.at[0], cisem.at[0]).wait()
    pltpu.sync_copy(col_hbm.at[pl.ds(ebase + _DCH_FULL * CH, _DTAIL)],
                    cidx.at[0, pl.ds(0, _DTAIL)])
    pltpu.sync_copy(ones_v.at[pl.ds(0, _DTAIL)],
                    deg_sh.at[cidx.at[0, pl.ds(0, _DTAIL)]], add=True)

    plsc.subcore_barrier()
    pltpu.sync_copy(deg_sh.at[pl.ds(rb, RPT)], out_hbm.at[c, pl.ds(rb, RPT)])


_deg_call = functools.partial(
    pl.kernel,
    out_type=jax.ShapeDtypeStruct((NC, NP), jnp.float32),
    mesh=plsc.VectorSubcoreMesh(**_MESH),
    scratch_types=[
        pltpu.VMEM((2, CH), jnp.int32),
        pltpu.VMEM((CH,), jnp.float32),
        pltpu.VMEM((RPT,), jnp.float32),
        pltpu.VMEM_SHARED((NP,), jnp.float32),
        pltpu.SemaphoreType.DMA((2,)),
    ],
)(_deg_body)


_EPT = E // NS
G = 2
_NCH = _EPT // CH
_NG = _NCH // G
_PTAIL = _EPT - _NCH * CH


def _prop_body(hs_hbm, row_hbm, col_hbm, out_hbm,
               ridx, cidx, rows, agg_sh, gsem, ssem, risem, cisem):
    c = lax.axis_index("c")
    s = lax.axis_index("s")
    ebase = s * _EPT
    rb = s * RPT
    hs_c = hs_hbm.at[c]

    pltpu.sync_copy(hs_hbm.at[c, pl.ds(rb, RPT)], agg_sh.at[pl.ds(rb, RPT)])
    plsc.subcore_barrier()

    def group(g, _):
        base = ebase + g * (G * CH)

        @pl.when(g > 0)
        def _wait_prev():
            for i in range(G):
                pltpu.make_async_copy(
                    rows.at[i], agg_sh.at[cidx.at[i]], ssem.at[i]).wait()

        for i in range(G):
            off = base + i * CH
            pltpu.async_copy(row_hbm.at[pl.ds(off, CH)], ridx.at[i],
                             risem.at[i])
            pltpu.async_copy(col_hbm.at[pl.ds(off, CH)], cidx.at[i],
                             cisem.at[i])
        for i in range(G):
            off = base + i * CH
            pltpu.make_async_copy(row_hbm.at[pl.ds(off, CH)], ridx.at[i],
                                  risem.at[i]).wait()
            pltpu.async_copy(hs_c.at[ridx.at[i]], rows.at[i], gsem.at[i])
        for i in range(G):
            off = base + i * CH
            pltpu.make_async_copy(col_hbm.at[pl.ds(off, CH)], cidx.at[i],
                                  cisem.at[i]).wait()
            pltpu.make_async_copy(hs_c.at[ridx.at[i]], rows.at[i],
                                  gsem.at[i]).wait()
            pltpu.async_copy(rows.at[i], agg_sh.at[cidx.at[i]], ssem.at[i],
                             add=True)
        return _

    lax.fori_loop(0, _NG, group, 0)
    for i in range(G):
        pltpu.make_async_copy(rows.at[i], agg_sh.at[cidx.at[i]],
                              ssem.at[i]).wait()

    toff = ebase + _NCH * CH
    pltpu.sync_copy(row_hbm.at[pl.ds(toff, _PTAIL)],
                    ridx.at[0, pl.ds(0, _PTAIL)])
    pltpu.sync_copy(col_hbm.at[pl.ds(toff, _PTAIL)],
                    cidx.at[0, pl.ds(0, _PTAIL)])
    pltpu.async_copy(hs_c.at[ridx.at[0, pl.ds(0, _PTAIL)]],
                     rows.at[0, pl.ds(0, _PTAIL)], gsem.at[0]).wait()
    pltpu.sync_copy(rows.at[0, pl.ds(0, _PTAIL)],
                    agg_sh.at[cidx.at[0, pl.ds(0, _PTAIL)]], add=True)

    plsc.subcore_barrier()
    pltpu.sync_copy(agg_sh.at[pl.ds(rb, RPT)], out_hbm.at[c, pl.ds(rb, RPT)])


_prop_call = functools.partial(
    pl.kernel,
    out_type=jax.ShapeDtypeStruct((NC, NP, HALF), jnp.float32),
    mesh=plsc.VectorSubcoreMesh(**_MESH),
    scratch_types=[
        pltpu.VMEM((G, CH), jnp.int32),
        pltpu.VMEM((G, CH), jnp.int32),
        pltpu.VMEM((G, CH, HALF), jnp.float32),
        pltpu.VMEM_SHARED((NP, HALF), jnp.float32),
        pltpu.SemaphoreType.DMA((G,)),
        pltpu.SemaphoreType.DMA((G,)),
        pltpu.SemaphoreType.DMA((G,)),
        pltpu.SemaphoreType.DMA((G,)),
    ],
)(_prop_body)


_R = 2048


def _stage1_body(x_ref, w0_ref, b0_ref, d0_ref, d1_ref,
                 x0_ref, hs_ref, dinv_ref):
    d = d0_ref[...] + d1_ref[...] + 1.0
    dinv = lax.rsqrt(d)
    h = jnp.maximum(
        jnp.dot(x_ref[...], w0_ref[...], preferred_element_type=jnp.float32)
        + b0_ref[...], 0.0)
    x0_ref[...] = h
    dinv_ref[...] = dinv
    hs = h * dinv
    hs_ref[0] = hs[:, :HALF]
    hs_ref[1] = hs[:, HALF:]


def _stage1(x_pad, W0, b0, d0, d1):
    return pl.pallas_call(
        _stage1_body,
        grid=(NP // _R,),
        in_specs=[
            pl.BlockSpec((_R, IN_C), lambda i: (i, 0)),
            pl.BlockSpec((IN_C, HID), lambda i: (0, 0)),
            pl.BlockSpec((1, HID), lambda i: (0, 0)),
            pl.BlockSpec((_R, 1), lambda i: (i, 0)),
            pl.BlockSpec((_R, 1), lambda i: (i, 0)),
        ],
        out_specs=[
            pl.BlockSpec((_R, HID), lambda i: (i, 0)),
            pl.BlockSpec((NC, _R, HALF), lambda i: (0, i, 0)),
            pl.BlockSpec((_R, 1), lambda i: (i, 0)),
        ],
        out_shape=[
            jax.ShapeDtypeStruct((NP, HID), jnp.float32),
            jax.ShapeDtypeStruct((NC, NP, HALF), jnp.float32),
            jax.ShapeDtypeStruct((NP, 1), jnp.float32),
        ],
    )(x_pad, W0, b0, d0, d1)


def _layer_body(beta, final, a_ref, x0_ref, dinv_ref, w_ref, g_ref, bb_ref,
                w1_ref, b1_ref, o_ref):
    dinv = dinv_ref[...]
    aggs = jnp.concatenate([a_ref[0], a_ref[1]], axis=1)
    t = (1.0 - ALPHA) * (aggs * dinv) + ALPHA * x0_ref[...]
    u = (1.0 - beta) * t + beta * jnp.dot(
        t, w_ref[...], preferred_element_type=jnp.float32)
    h = jnp.maximum(u * (BN_SCALE * g_ref[...]) + bb_ref[...], 0.0)
    if final:
        o_ref[...] = jnp.dot(
            h, w1_ref[...], preferred_element_type=jnp.float32) + b1_ref[...]
    else:
        hs = h * dinv
        o_ref[0] = hs[:, :HALF]
        o_ref[1] = hs[:, HALF:]


def _layer(i, aggs, x0, dinv, W, gamma, bnb, W1, b1):
    beta = float(np.log(THETA / (i + 1) + 1.0))
    final = i == L - 1
    if final:
        out_spec = pl.BlockSpec((_R, OUT_C), lambda i: (i, 0))
        out_shape = jax.ShapeDtypeStruct((NP, OUT_C), jnp.float32)
    else:
        out_spec = pl.BlockSpec((NC, _R, HALF), lambda i: (0, i, 0))
        out_shape = jax.ShapeDtypeStruct((NC, NP, HALF), jnp.float32)
    return pl.pallas_call(
        functools.partial(_layer_body, beta, final),
        grid=(NP // _R,),
        in_specs=[
            pl.BlockSpec((NC, _R, HALF), lambda i: (0, i, 0)),
            pl.BlockSpec((_R, HID), lambda i: (i, 0)),
            pl.BlockSpec((_R, 1), lambda i: (i, 0)),
            pl.BlockSpec((HID, HID), lambda i: (0, 0)),
            pl.BlockSpec((1, HID), lambda i: (0, 0)),
            pl.BlockSpec((1, HID), lambda i: (0, 0)),
            pl.BlockSpec((HID, OUT_C), lambda i: (0, 0)),
            pl.BlockSpec((1, OUT_C), lambda i: (0, 0)),
        ],
        out_specs=out_spec,
        out_shape=out_shape,
    )(aggs, x0, dinv, W, gamma, bnb, W1, b1)


@jax.jit
def kernel(x, edge_index, W0, b0, W1, b1, conv_ws, bn_gamma, bn_beta):
    row = edge_index[0]
    col = edge_index[1]
    x_pad = jnp.pad(x, ((0, NP - N), (0, 0)))

    deg = _deg_call(col)
    d0 = deg[0].reshape(NP, 1)
    d1 = deg[1].reshape(NP, 1)

    x0, hs, dinv = _stage1(x_pad, W0, b0.reshape(1, HID), d0, d1)

    b1r = b1.reshape(1, OUT_C)
    res = None
    for i in range(L):
        aggs = _prop_call(hs, row, col)
        res = _layer(i, aggs, x0, dinv, conv_ws[i],
                     bn_gamma[i].reshape(1, HID), bn_beta[i].reshape(1, HID),
                     W1, b1r)
        hs = res
    return res[:N]


# speedup vs baseline: 16.6008x; 1.2241x over previous
import functools

import numpy as np
import jax
import jax.numpy as jnp
from jax import lax
from jax.experimental import pallas as pl
from jax.experimental.pallas import tpu as pltpu
from jax.experimental.pallas import tpu_sc as plsc

N = 10000
NP = 10240
E = 320000
IN_C = 128
HID = 256
HALF = 128
OUT_C = 128
L = 4
ALPHA = 0.1
THETA = 0.5
BN_SCALE = float(1.0 / np.sqrt(1.0 + 1e-5))

NC, NS = 2, 16
NW = NC * NS
RPT = NP // NS
CH = 128

_MESH = dict(core_axis_name="c", subcore_axis_name="s", num_cores=NC,
             num_subcores=NS)


_EPW = E // NW
_DCH_FULL = _EPW // CH
_DTAIL = _EPW - _DCH_FULL * CH


def _deg_body(col_hbm, out_hbm, cidx, ones_v, zeros_v, deg_sh, cisem):
    c = lax.axis_index("c")
    s = lax.axis_index("s")
    wid = s * NC + c
    ebase = wid * _EPW
    rb = s * RPT

    def fill(ref, n, val):
        def body(i, _):
            ref[pl.ds(i * 16, 16)] = jnp.full((16,), val, jnp.float32)
            return _
        lax.fori_loop(0, n // 16, body, 0)

    fill(ones_v, CH, 1.0)
    fill(zeros_v, RPT, 0.0)
    pltpu.sync_copy(zeros_v, deg_sh.at[pl.ds(rb, RPT)])
    plsc.subcore_barrier()

    pltpu.async_copy(col_hbm.at[pl.ds(ebase, CH)], cidx.at[0], cisem.at[0])

    def pair(m, _):
        for sl in range(2):
            j = 2 * m + sl
            off = ebase + j * CH
            pltpu.make_async_copy(
                col_hbm.at[pl.ds(off, CH)], cidx.at[sl], cisem.at[sl]).wait()
            nxt = jnp.minimum(j + 1, _DCH_FULL - 1)
            pltpu.async_copy(col_hbm.at[pl.ds(ebase + nxt * CH, CH)],
                             cidx.at[1 - sl], cisem.at[1 - sl])
            pltpu.sync_copy(ones_v, deg_sh.at[cidx.at[sl]], add=True)
        return _

    lax.fori_loop(0, _DCH_FULL // 2, pair, 0)
    pltpu.make_async_copy(col_hbm.at[pl.ds(ebase + (_DCH_FULL - 1) * CH, CH)],
                          cidx.at[0], cisem.at[0]).wait()
    pltpu.sync_copy(col_hbm.at[pl.ds(ebase + _DCH_FULL * CH, _DTAIL)],
                    cidx.at[0, pl.ds(0, _DTAIL)])
    pltpu.sync_copy(ones_v.at[pl.ds(0, _DTAIL)],
                    deg_sh.at[cidx.at[0, pl.ds(0, _DTAIL)]], add=True)

    plsc.subcore_barrier()
    pltpu.sync_copy(deg_sh.at[pl.ds(rb, RPT)], out_hbm.at[c, pl.ds(rb, RPT)])


_deg_call = functools.partial(
    pl.kernel,
    out_type=jax.ShapeDtypeStruct((NC, NP), jnp.float32),
    mesh=plsc.VectorSubcoreMesh(**_MESH),
    scratch_types=[
        pltpu.VMEM((2, CH), jnp.int32),
        pltpu.VMEM((CH,), jnp.float32),
        pltpu.VMEM((RPT,), jnp.float32),
        pltpu.VMEM_SHARED((NP,), jnp.float32),
        pltpu.SemaphoreType.DMA((2,)),
    ],
)(_deg_body)


_NCHUNK = E // CH
_CPT = _NCHUNK // NS
_XTRA = _NCHUNK - _CPT * NS
_SG = 13
_NSG = _CPT // _SG


def _prop_body(hs_hbm, row_hbm, col_hbm, out_hbm,
               ridx, cidx, rows, agg_sh, gsem, ssem, risem, cisem):
    c = lax.axis_index("c")
    s = lax.axis_index("s")
    cbase = s * _CPT
    rb = s * RPT
    hs_c = hs_hbm.at[c]

    pltpu.sync_copy(hs_hbm.at[c, pl.ds(rb, RPT)], agg_sh.at[pl.ds(rb, RPT)])
    plsc.subcore_barrier()

    def idx_start(sg):
        sl = sg % 3
        for k in range(_SG):
            off = (cbase + sg * _SG + k) * CH
            pltpu.async_copy(row_hbm.at[pl.ds(off, CH)], ridx.at[sl, k],
                             risem.at[sl])
            pltpu.async_copy(col_hbm.at[pl.ds(off, CH)], cidx.at[sl, k],
                             cisem.at[sl])

    def idx_wait(sg):
        sl = sg % 3
        for k in range(_SG):
            off = (cbase + sg * _SG + k) * CH
            pltpu.make_async_copy(row_hbm.at[pl.ds(off, CH)],
                                  ridx.at[sl, k], risem.at[sl]).wait()
            pltpu.make_async_copy(col_hbm.at[pl.ds(off, CH)],
                                  cidx.at[sl, k], cisem.at[sl]).wait()

    def g_start(j, r):
        pltpu.async_copy(hs_c.at[ridx.at[(j // _SG) % 3, j % _SG]],
                         rows.at[r], gsem.at[r])

    def g_wait(j, r):
        pltpu.make_async_copy(hs_c.at[ridx.at[(j // _SG) % 3, j % _SG]],
                              rows.at[r], gsem.at[r]).wait()

    def s_start(j, r):
        pltpu.async_copy(rows.at[r], agg_sh.at[cidx.at[(j // _SG) % 3,
                                                       j % _SG]],
                         ssem.at[r], add=True)

    def s_wait(j, r):
        pltpu.make_async_copy(rows.at[r],
                              agg_sh.at[cidx.at[(j // _SG) % 3, j % _SG]],
                              ssem.at[r]).wait()

    idx_start(0)
    idx_start(1)
    idx_wait(0)
    g_start(0, 0)

    def chunk(j, carry):
        sg = j // _SG
        k = j - sg * _SG
        r = lax.rem(j, 2)
        g_wait(j, r)

        @pl.when(j >= 1)
        def _():
            s_wait(j - 1, 1 - r)

        @pl.when((k == 1) & (sg < _NSG - 2))
        def _():
            idx_start(sg + 2)

        @pl.when((k == _SG - 1) & (sg < _NSG - 1))
        def _():
            idx_wait(sg + 1)

        @pl.when(j < _CPT - 1)
        def _():
            g_start(j + 1, 1 - r)

        s_start(j, r)
        return carry

    lax.fori_loop(0, _CPT, chunk, 0)
    s_wait(_CPT - 1, (_CPT - 1) % 2)

    @pl.when(s < _XTRA)
    def _extra():
        xoff = (NS * _CPT + s) * CH
        pltpu.sync_copy(row_hbm.at[pl.ds(xoff, CH)], ridx.at[0, 0])
        pltpu.sync_copy(col_hbm.at[pl.ds(xoff, CH)], cidx.at[0, 0])
        pltpu.async_copy(hs_c.at[ridx.at[0, 0]], rows.at[0],
                         gsem.at[0]).wait()
        pltpu.sync_copy(rows.at[0], agg_sh.at[cidx.at[0, 0]], add=True)

    plsc.subcore_barrier()
    pltpu.sync_copy(agg_sh.at[pl.ds(rb, RPT)], out_hbm.at[c, pl.ds(rb, RPT)])


_prop_call = functools.partial(
    pl.kernel,
    out_type=jax.ShapeDtypeStruct((NC, NP, HALF), jnp.float32),
    mesh=plsc.VectorSubcoreMesh(**_MESH),
    scratch_types=[
        pltpu.VMEM((3, _SG, CH), jnp.int32),
        pltpu.VMEM((3, _SG, CH), jnp.int32),
        pltpu.VMEM((2, CH, HALF), jnp.float32),
        pltpu.VMEM_SHARED((NP, HALF), jnp.float32),
        pltpu.SemaphoreType.DMA((2,)),
        pltpu.SemaphoreType.DMA((2,)),
        pltpu.SemaphoreType.DMA((3,)),
        pltpu.SemaphoreType.DMA((3,)),
    ],
)(_prop_body)


_R = 2048


def _stage1_body(x_ref, w0_ref, b0_ref, d0_ref, d1_ref,
                 x0_ref, hs_ref, dinv_ref):
    d = d0_ref[...] + d1_ref[...] + 1.0
    dinv = lax.rsqrt(d)
    h = jnp.maximum(
        jnp.dot(x_ref[...], w0_ref[...], preferred_element_type=jnp.float32)
        + b0_ref[...], 0.0)
    x0_ref[...] = h
    dinv_ref[...] = dinv
    hs = h * dinv
    hs_ref[0] = hs[:, :HALF]
    hs_ref[1] = hs[:, HALF:]


def _stage1(x_pad, W0, b0, d0, d1):
    return pl.pallas_call(
        _stage1_body,
        grid=(NP // _R,),
        in_specs=[
            pl.BlockSpec((_R, IN_C), lambda i: (i, 0)),
            pl.BlockSpec((IN_C, HID), lambda i: (0, 0)),
            pl.BlockSpec((1, HID), lambda i: (0, 0)),
            pl.BlockSpec((_R, 1), lambda i: (i, 0)),
            pl.BlockSpec((_R, 1), lambda i: (i, 0)),
        ],
        out_specs=[
            pl.BlockSpec((_R, HID), lambda i: (i, 0)),
            pl.BlockSpec((NC, _R, HALF), lambda i: (0, i, 0)),
            pl.BlockSpec((_R, 1), lambda i: (i, 0)),
        ],
        out_shape=[
            jax.ShapeDtypeStruct((NP, HID), jnp.float32),
            jax.ShapeDtypeStruct((NC, NP, HALF), jnp.float32),
            jax.ShapeDtypeStruct((NP, 1), jnp.float32),
        ],
    )(x_pad, W0, b0, d0, d1)


def _layer_body(beta, final, a_ref, x0_ref, dinv_ref, w_ref, g_ref, bb_ref,
                w1_ref, b1_ref, o_ref):
    dinv = dinv_ref[...]
    aggs = jnp.concatenate([a_ref[0], a_ref[1]], axis=1)
    t = (1.0 - ALPHA) * (aggs * dinv) + ALPHA * x0_ref[...]
    u = (1.0 - beta) * t + beta * jnp.dot(
        t, w_ref[...], preferred_element_type=jnp.float32)
    h = jnp.maximum(u * (BN_SCALE * g_ref[...]) + bb_ref[...], 0.0)
    if final:
        o_ref[...] = jnp.dot(
            h, w1_ref[...], preferred_element_type=jnp.float32) + b1_ref[...]
    else:
        hs = h * dinv
        o_ref[0] = hs[:, :HALF]
        o_ref[1] = hs[:, HALF:]


def _layer(i, aggs, x0, dinv, W, gamma, bnb, W1, b1):
    beta = float(np.log(THETA / (i + 1) + 1.0))
    final = i == L - 1
    if final:
        out_spec = pl.BlockSpec((_R, OUT_C), lambda i: (i, 0))
        out_shape = jax.ShapeDtypeStruct((NP, OUT_C), jnp.float32)
    else:
        out_spec = pl.BlockSpec((NC, _R, HALF), lambda i: (0, i, 0))
        out_shape = jax.ShapeDtypeStruct((NC, NP, HALF), jnp.float32)
    return pl.pallas_call(
        functools.partial(_layer_body, beta, final),
        grid=(NP // _R,),
        in_specs=[
            pl.BlockSpec((NC, _R, HALF), lambda i: (0, i, 0)),
            pl.BlockSpec((_R, HID), lambda i: (i, 0)),
            pl.BlockSpec((_R, 1), lambda i: (i, 0)),
            pl.BlockSpec((HID, HID), lambda i: (0, 0)),
            pl.BlockSpec((1, HID), lambda i: (0, 0)),
            pl.BlockSpec((1, HID), lambda i: (0, 0)),
            pl.BlockSpec((HID, OUT_C), lambda i: (0, 0)),
            pl.BlockSpec((1, OUT_C), lambda i: (0, 0)),
        ],
        out_specs=out_spec,
        out_shape=out_shape,
    )(aggs, x0, dinv, W, gamma, bnb, W1, b1)


@jax.jit
def kernel(x, edge_index, W0, b0, W1, b1, conv_ws, bn_gamma, bn_beta):
    row = edge_index[0]
    col = edge_index[1]
    x_pad = jnp.pad(x, ((0, NP - N), (0, 0)))

    deg = _deg_call(col)
    d0 = deg[0].reshape(NP, 1)
    d1 = deg[1].reshape(NP, 1)

    x0, hs, dinv = _stage1(x_pad, W0, b0.reshape(1, HID), d0, d1)

    b1r = b1.reshape(1, OUT_C)
    res = None
    for i in range(L):
        aggs = _prop_call(hs, row, col)
        res = _layer(i, aggs, x0, dinv, conv_ws[i],
                     bn_gamma[i].reshape(1, HID), bn_beta[i].reshape(1, HID),
                     W1, b1r)
        hs = res
    return res[:N]


# speedup vs baseline: 17.8952x; 1.0780x over previous
import functools

import numpy as np
import jax
import jax.numpy as jnp
from jax import lax
from jax.experimental import pallas as pl
from jax.experimental.pallas import tpu as pltpu
from jax.experimental.pallas import tpu_sc as plsc

N = 10000
NP = 10240
E = 320000
IN_C = 128
HID = 256
HALF = 128
OUT_C = 128
L = 4
ALPHA = 0.1
THETA = 0.5
BN_SCALE = float(1.0 / np.sqrt(1.0 + 1e-5))

NC, NS = 2, 16
NW = NC * NS
RPT = NP // NS
CH = 128

_MESH = dict(core_axis_name="c", subcore_axis_name="s", num_cores=NC,
             num_subcores=NS)


_EPW = E // NW
_DCH_FULL = _EPW // CH
_DTAIL = _EPW - _DCH_FULL * CH


def _deg_body(col_hbm, out_hbm, cidx, ones_v, zeros_v, deg_sh, cisem):
    c = lax.axis_index("c")
    s = lax.axis_index("s")
    wid = s * NC + c
    ebase = wid * _EPW
    rb = s * RPT

    def fill(ref, n, val):
        def body(i, _):
            ref[pl.ds(i * 16, 16)] = jnp.full((16,), val, jnp.float32)
            return _
        lax.fori_loop(0, n // 16, body, 0)

    fill(ones_v, CH, 1.0)
    fill(zeros_v, RPT, 0.0)
    pltpu.sync_copy(zeros_v, deg_sh.at[pl.ds(rb, RPT)])
    plsc.subcore_barrier()

    pltpu.async_copy(col_hbm.at[pl.ds(ebase, CH)], cidx.at[0], cisem.at[0])

    def pair(m, _):
        for sl in range(2):
            j = 2 * m + sl
            off = ebase + j * CH
            pltpu.make_async_copy(
                col_hbm.at[pl.ds(off, CH)], cidx.at[sl], cisem.at[sl]).wait()
            nxt = jnp.minimum(j + 1, _DCH_FULL - 1)
            pltpu.async_copy(col_hbm.at[pl.ds(ebase + nxt * CH, CH)],
                             cidx.at[1 - sl], cisem.at[1 - sl])
            pltpu.sync_copy(ones_v, deg_sh.at[cidx.at[sl]], add=True)
        return _

    lax.fori_loop(0, _DCH_FULL // 2, pair, 0)
    pltpu.make_async_copy(col_hbm.at[pl.ds(ebase + (_DCH_FULL - 1) * CH, CH)],
                          cidx.at[0], cisem.at[0]).wait()
    pltpu.sync_copy(col_hbm.at[pl.ds(ebase + _DCH_FULL * CH, _DTAIL)],
                    cidx.at[0, pl.ds(0, _DTAIL)])
    pltpu.sync_copy(ones_v.at[pl.ds(0, _DTAIL)],
                    deg_sh.at[cidx.at[0, pl.ds(0, _DTAIL)]], add=True)

    plsc.subcore_barrier()
    pltpu.sync_copy(deg_sh.at[pl.ds(rb, RPT)], out_hbm.at[c, pl.ds(rb, RPT)])


_deg_call = functools.partial(
    pl.kernel,
    out_type=jax.ShapeDtypeStruct((NC, NP), jnp.float32),
    mesh=plsc.VectorSubcoreMesh(**_MESH),
    scratch_types=[
        pltpu.VMEM((2, CH), jnp.int32),
        pltpu.VMEM((CH,), jnp.float32),
        pltpu.VMEM((RPT,), jnp.float32),
        pltpu.VMEM_SHARED((NP,), jnp.float32),
        pltpu.SemaphoreType.DMA((2,)),
    ],
)(_deg_body)


PCH = 64
_NCHUNK = E // PCH
_CPT = _NCHUNK // NS
_XTRA = _NCHUNK - _CPT * NS
_SG = 13
_NSG = _CPT // _SG
_D = 4


def _prop_body(hs_hbm, row_hbm, col_hbm, out_hbm,
               ridx, cidx, rows, agg_sh, gsem, ssem, risem, cisem):
    c = lax.axis_index("c")
    s = lax.axis_index("s")
    cbase = s * _CPT
    rb = s * RPT
    hs_c = hs_hbm.at[c]

    pltpu.sync_copy(hs_hbm.at[c, pl.ds(rb, RPT)], agg_sh.at[pl.ds(rb, RPT)])
    plsc.subcore_barrier()

    def idx_start(sg):
        sl = sg % 3
        for k in range(_SG):
            off = (cbase + sg * _SG + k) * PCH
            pltpu.async_copy(row_hbm.at[pl.ds(off, PCH)], ridx.at[sl, k],
                             risem.at[sl])
            pltpu.async_copy(col_hbm.at[pl.ds(off, PCH)], cidx.at[sl, k],
                             cisem.at[sl])

    def idx_wait(sg):
        sl = sg % 3
        for k in range(_SG):
            off = (cbase + sg * _SG + k) * PCH
            pltpu.make_async_copy(row_hbm.at[pl.ds(off, PCH)],
                                  ridx.at[sl, k], risem.at[sl]).wait()
            pltpu.make_async_copy(col_hbm.at[pl.ds(off, PCH)],
                                  cidx.at[sl, k], cisem.at[sl]).wait()

    def g_start(j):
        r = lax.rem(j, _D)
        pltpu.async_copy(hs_c.at[ridx.at[(j // _SG) % 3, j % _SG]],
                         rows.at[r], gsem.at[r])

    def g_wait(j):
        r = lax.rem(j, _D)
        pltpu.make_async_copy(hs_c.at[ridx.at[(j // _SG) % 3, j % _SG]],
                              rows.at[r], gsem.at[r]).wait()

    def s_start(j):
        r = lax.rem(j, _D)
        pltpu.async_copy(rows.at[r], agg_sh.at[cidx.at[(j // _SG) % 3,
                                                       j % _SG]],
                         ssem.at[r], add=True)

    def s_wait(j):
        r = lax.rem(j, _D)
        pltpu.make_async_copy(rows.at[r],
                              agg_sh.at[cidx.at[(j // _SG) % 3, j % _SG]],
                              ssem.at[r]).wait()

    idx_start(0)
    idx_start(1)
    idx_wait(0)
    g_start(0)
    g_start(1)

    def chunk(j, carry):
        sg = j // _SG
        k = j - sg * _SG
        g_wait(j)

        @pl.when(j >= 2)
        def _():
            s_wait(j - 2)

        @pl.when((k == 2) & (sg < _NSG - 2))
        def _():
            idx_start(sg + 2)

        @pl.when((k == _SG - 1) & (sg < _NSG - 1))
        def _():
            idx_wait(sg + 1)

        @pl.when(j < _CPT - 2)
        def _():
            g_start(j + 2)

        s_start(j)
        return carry

    lax.fori_loop(0, _CPT, chunk, 0)
    for j in range(_CPT - 2, _CPT):
        s_wait(j)

    @pl.when(s < _XTRA)
    def _extra():
        xoff = (NS * _CPT + s) * PCH
        pltpu.sync_copy(row_hbm.at[pl.ds(xoff, PCH)], ridx.at[0, 0])
        pltpu.sync_copy(col_hbm.at[pl.ds(xoff, PCH)], cidx.at[0, 0])
        pltpu.async_copy(hs_c.at[ridx.at[0, 0]], rows.at[0],
                         gsem.at[0]).wait()
        pltpu.sync_copy(rows.at[0], agg_sh.at[cidx.at[0, 0]], add=True)

    plsc.subcore_barrier()
    pltpu.sync_copy(agg_sh.at[pl.ds(rb, RPT)], out_hbm.at[c, pl.ds(rb, RPT)])


_prop_call = functools.partial(
    pl.kernel,
    out_type=jax.ShapeDtypeStruct((NC, NP, HALF), jnp.float32),
    mesh=plsc.VectorSubcoreMesh(**_MESH),
    scratch_types=[
        pltpu.VMEM((3, _SG, PCH), jnp.int32),
        pltpu.VMEM((3, _SG, PCH), jnp.int32),
        pltpu.VMEM((_D, PCH, HALF), jnp.float32),
        pltpu.VMEM_SHARED((NP, HALF), jnp.float32),
        pltpu.SemaphoreType.DMA((_D,)),
        pltpu.SemaphoreType.DMA((_D,)),
        pltpu.SemaphoreType.DMA((3,)),
        pltpu.SemaphoreType.DMA((3,)),
    ],
)(_prop_body)


_R = 2048


def _stage1_body(x_ref, w0_ref, b0_ref, d0_ref, d1_ref,
                 x0_ref, hs_ref, dinv_ref):
    d = d0_ref[...] + d1_ref[...] + 1.0
    dinv = lax.rsqrt(d)
    h = jnp.maximum(
        jnp.dot(x_ref[...], w0_ref[...], preferred_element_type=jnp.float32)
        + b0_ref[...], 0.0)
    x0_ref[...] = h
    dinv_ref[...] = dinv
    hs = h * dinv
    hs_ref[0] = hs[:, :HALF]
    hs_ref[1] = hs[:, HALF:]


def _stage1(x_pad, W0, b0, d0, d1):
    return pl.pallas_call(
        _stage1_body,
        grid=(NP // _R,),
        in_specs=[
            pl.BlockSpec((_R, IN_C), lambda i: (i, 0)),
            pl.BlockSpec((IN_C, HID), lambda i: (0, 0)),
            pl.BlockSpec((1, HID), lambda i: (0, 0)),
            pl.BlockSpec((_R, 1), lambda i: (i, 0)),
            pl.BlockSpec((_R, 1), lambda i: (i, 0)),
        ],
        out_specs=[
            pl.BlockSpec((_R, HID), lambda i: (i, 0)),
            pl.BlockSpec((NC, _R, HALF), lambda i: (0, i, 0)),
            pl.BlockSpec((_R, 1), lambda i: (i, 0)),
        ],
        out_shape=[
            jax.ShapeDtypeStruct((NP, HID), jnp.float32),
            jax.ShapeDtypeStruct((NC, NP, HALF), jnp.float32),
            jax.ShapeDtypeStruct((NP, 1), jnp.float32),
        ],
    )(x_pad, W0, b0, d0, d1)


def _layer_body(beta, final, a_ref, x0_ref, dinv_ref, w_ref, g_ref, bb_ref,
                w1_ref, b1_ref, o_ref):
    dinv = dinv_ref[...]
    aggs = jnp.concatenate([a_ref[0], a_ref[1]], axis=1)
    t = (1.0 - ALPHA) * (aggs * dinv) + ALPHA * x0_ref[...]
    u = (1.0 - beta) * t + beta * jnp.dot(
        t, w_ref[...], preferred_element_type=jnp.float32)
    h = jnp.maximum(u * (BN_SCALE * g_ref[...]) + bb_ref[...], 0.0)
    if final:
        o_ref[...] = jnp.dot(
            h, w1_ref[...], preferred_element_type=jnp.float32) + b1_ref[...]
    else:
        hs = h * dinv
        o_ref[0] = hs[:, :HALF]
        o_ref[1] = hs[:, HALF:]


def _layer(i, aggs, x0, dinv, W, gamma, bnb, W1, b1):
    beta = float(np.log(THETA / (i + 1) + 1.0))
    final = i == L - 1
    if final:
        out_spec = pl.BlockSpec((_R, OUT_C), lambda i: (i, 0))
        out_shape = jax.ShapeDtypeStruct((NP, OUT_C), jnp.float32)
    else:
        out_spec = pl.BlockSpec((NC, _R, HALF), lambda i: (0, i, 0))
        out_shape = jax.ShapeDtypeStruct((NC, NP, HALF), jnp.float32)
    return pl.pallas_call(
        functools.partial(_layer_body, beta, final),
        grid=(NP // _R,),
        in_specs=[
            pl.BlockSpec((NC, _R, HALF), lambda i: (0, i, 0)),
            pl.BlockSpec((_R, HID), lambda i: (i, 0)),
            pl.BlockSpec((_R, 1), lambda i: (i, 0)),
            pl.BlockSpec((HID, HID), lambda i: (0, 0)),
            pl.BlockSpec((1, HID), lambda i: (0, 0)),
            pl.BlockSpec((1, HID), lambda i: (0, 0)),
            pl.BlockSpec((HID, OUT_C), lambda i: (0, 0)),
            pl.BlockSpec((1, OUT_C), lambda i: (0, 0)),
        ],
        out_specs=out_spec,
        out_shape=out_shape,
    )(aggs, x0, dinv, W, gamma, bnb, W1, b1)


@jax.jit
def kernel(x, edge_index, W0, b0, W1, b1, conv_ws, bn_gamma, bn_beta):
    row = edge_index[0]
    col = edge_index[1]
    x_pad = jnp.pad(x, ((0, NP - N), (0, 0)))

    deg = _deg_call(col)
    d0 = deg[0].reshape(NP, 1)
    d1 = deg[1].reshape(NP, 1)

    x0, hs, dinv = _stage1(x_pad, W0, b0.reshape(1, HID), d0, d1)

    b1r = b1.reshape(1, OUT_C)
    res = None
    for i in range(L):
        aggs = _prop_call(hs, row, col)
        res = _layer(i, aggs, x0, dinv, conv_ws[i],
                     bn_gamma[i].reshape(1, HID), bn_beta[i].reshape(1, HID),
                     W1, b1r)
        hs = res
    return res[:N]
